# baseline (speedup 1.0000x reference)
import functools
import math

import jax
import jax.numpy as jnp
from jax.experimental import pallas as pl
from jax.experimental.pallas import tpu as pltpu
from jax.experimental.pallas import tpu_sc as plsc

EMB_DIM = 32
SCALE = math.sqrt(float(EMB_DIM))
LANES = 16
N_L = 200
N_B = 4096
W = 512


def _build_gather():
    mesh = plsc.VectorSubcoreMesh(core_axis_name="c", subcore_axis_name="s")
    cp = pltpu.CompilerParams(
        needs_layout_passes=False, use_tc_tiling_on_sc=False
    )
    n = N_L * N_B

    @functools.partial(
        pl.kernel,
        out_type=jax.ShapeDtypeStruct((n, EMB_DIM), jnp.float32),
        mesh=mesh,
        compiler_params=cp,
        scratch_types=[pltpu.VMEM((W,), jnp.int32)],
    )
    def gather_kernel(table_hbm, idx_hbm, out_hbm, i4_scr):
        def body(i_vmem, o_vmem):
            @pl.loop(0, W, step=LANES)
            def _(j):
                i4_scr[pl.ds(j, LANES)] = i_vmem[0, pl.ds(j, LANES)] * 4

            pltpu.sync_copy(table_hbm.at[i4_scr], o_vmem)

        pltpu.emit_pipeline(
            body,
            grid=(n // W,),
            in_specs=[pl.BlockSpec((1, W), lambda i: (0, i))],
            out_specs=[pl.BlockSpec((W, EMB_DIM), lambda i: (i, 0))],
            core_axis_name=("c", "s"),
            dimension_semantics=(pltpu.PARALLEL,),
        )(idx_hbm, out_hbm)

    return gather_kernel


def _finish_body(g_ref, tok_ref, o_ref):
    x = g_ref[0]
    tok = tok_ref[0, 0]
    scale = jnp.where(tok == 1, 0.0, SCALE).astype(jnp.float32)
    m = x.reshape(W // 4, EMB_DIM).T
    y = m * scale[None, :]
    o_ref[0, :, 0, :, :] = y.reshape(4, 8, W // 4)


def _build_finish():
    nbt = N_B // (W // 4)

    return pl.pallas_call(
        _finish_body,
        grid=(N_L, nbt),
        in_specs=[
            pl.BlockSpec((1, 32, W // 4), lambda l, t: (l, t, 0)),
            pl.BlockSpec((1, 1, W // 4), lambda l, t: (l * nbt + t, 0, 0)),
        ],
        out_specs=pl.BlockSpec(
            (1, 4, 1, 8, W // 4), lambda l, t: (l, 0, t, 0, 0)
        ),
        out_shape=jax.ShapeDtypeStruct((N_L, 4, nbt, 8, W // 4), jnp.float32),
        compiler_params=pltpu.CompilerParams(
            dimension_semantics=("parallel", "parallel")
        ),
    )


def kernel(tokens, table):
    n = N_L * N_B
    idx = tokens.T.reshape(1, n).astype(jnp.int32)
    tok3 = tokens.T.reshape(N_L * 32, 1, 128).astype(jnp.int32)
    table_pad = jnp.pad(table, ((0, 0), (0, 96))).reshape(4 * 1000000, EMB_DIM)
    g = _build_gather()(table_pad, idx)
    g4 = g.reshape(N_L, 1024, 128)
    out5 = _build_finish()(g4, tok3)
    return out5.transpose(2, 4, 0, 1, 3).reshape(N_B, N_L, EMB_DIM)


# speedup vs baseline: 2.7825x; 2.7825x over previous
import functools
import math

import jax
import jax.numpy as jnp
from jax.experimental import pallas as pl
from jax.experimental.pallas import tpu as pltpu
from jax.experimental.pallas import tpu_sc as plsc

EMB_DIM = 32
SCALE = math.sqrt(float(EMB_DIM))
LANES = 16
N_L = 200
N_B = 4096
W = 512


def _build_gather():
    mesh = plsc.VectorSubcoreMesh(core_axis_name="c", subcore_axis_name="s")
    cp = pltpu.CompilerParams(
        needs_layout_passes=False, use_tc_tiling_on_sc=False
    )
    n = N_L * N_B

    @functools.partial(
        pl.kernel,
        out_type=jax.ShapeDtypeStruct((n, EMB_DIM), jnp.float32),
        mesh=mesh,
        compiler_params=cp,
        scratch_types=[pltpu.VMEM((W,), jnp.int32)],
    )
    def gather_kernel(table_hbm, idx_hbm, out_hbm, i4_scr):
        def body(i_vmem, o_vmem):
            @pl.loop(0, W, step=LANES)
            def _(j):
                i4_scr[pl.ds(j, LANES)] = i_vmem[0, pl.ds(j, LANES)] * 4

            pltpu.sync_copy(table_hbm.at[i4_scr], o_vmem)

        pltpu.emit_pipeline(
            body,
            grid=(n // W,),
            in_specs=[pl.BlockSpec((1, W), lambda i: (0, i))],
            out_specs=[pl.BlockSpec((W, EMB_DIM), lambda i: (i, 0))],
            core_axis_name=("c", "s"),
            dimension_semantics=(pltpu.PARALLEL,),
        )(idx_hbm, out_hbm)

    return gather_kernel


def _finish_body(g_ref, tok_ref, o_ref):
    x = g_ref[0]
    y = x.T
    tok = tok_ref[0]
    scale = jnp.where(tok == 1, 0.0, SCALE).astype(jnp.float32)
    for t in range(8):
        blk = y[:, 128 * t : 128 * (t + 1)] * scale[t][None, :]
        o_ref[0, :, t, :, :] = blk.reshape(4, 8, 128)


def _build_finish():
    return pl.pallas_call(
        _finish_body,
        grid=(N_L, 4),
        in_specs=[
            pl.BlockSpec((1, 1024, EMB_DIM), lambda l, t: (l, t, 0)),
            pl.BlockSpec((1, 8, 128), lambda l, t: (l, t, 0)),
        ],
        out_specs=pl.BlockSpec(
            (1, 4, 8, 8, 128), lambda l, t: (l, 0, t, 0, 0)
        ),
        out_shape=jax.ShapeDtypeStruct((N_L, 4, 32, 8, 128), jnp.float32),
        compiler_params=pltpu.CompilerParams(
            dimension_semantics=("parallel", "parallel")
        ),
    )


def kernel(tokens, table):
    n = N_L * N_B
    idx = tokens.T.reshape(1, n).astype(jnp.int32)
    tok3 = tokens.T.reshape(N_L, 32, 128).astype(jnp.int32)
    table_pad = jnp.pad(table, ((0, 0), (0, 96))).reshape(4 * 1000000, EMB_DIM)
    g = _build_gather()(table_pad, idx)
    g3 = g.reshape(N_L, N_B, EMB_DIM)
    out5 = _build_finish()(g3, tok3)
    return out5.transpose(2, 4, 0, 1, 3).reshape(N_B, N_L, EMB_DIM)


# speedup vs baseline: 3.4424x; 1.2372x over previous
import functools
import math

import jax
import jax.numpy as jnp
from jax.experimental import pallas as pl
from jax.experimental.pallas import tpu as pltpu
from jax.experimental.pallas import tpu_sc as plsc

EMB_DIM = 32
SCALE = math.sqrt(float(EMB_DIM))
LANES = 16
N_L = 200
N_B = 4096
W = 512


def _build_gather():
    mesh = plsc.VectorSubcoreMesh(core_axis_name="c", subcore_axis_name="s")
    cp = pltpu.CompilerParams(
        needs_layout_passes=False, use_tc_tiling_on_sc=False
    )
    n = N_L * N_B

    @functools.partial(
        pl.kernel,
        out_type=jax.ShapeDtypeStruct((n, 4, EMB_DIM), jnp.float32),
        mesh=mesh,
        compiler_params=cp,
        scratch_types=[pltpu.VMEM((W,), jnp.int32)],
    )
    def gather_kernel(table_hbm, idx_hbm, out_hbm, i4_scr):
        def body(i_vmem, o_vmem):
            @pl.loop(0, W, step=LANES)
            def _(j):
                i4_scr[pl.ds(j, LANES)] = i_vmem[0, pl.ds(j, LANES)] * 4

            pltpu.sync_copy(table_hbm.at[i4_scr], o_vmem.at[:, 0, :])

        pltpu.emit_pipeline(
            body,
            grid=(n // W,),
            in_specs=[pl.BlockSpec((1, W), lambda i: (0, i))],
            out_specs=[pl.BlockSpec((W, 1, EMB_DIM), lambda i: (i, 0, 0))],
            core_axis_name=("c", "s"),
            dimension_semantics=(pltpu.PARALLEL,),
        )(idx_hbm, out_hbm)

    return gather_kernel


def _finish_body(g_ref, tok_ref, o_ref):
    x = g_ref[0][:, 0:EMB_DIM]
    y = x.T
    tok = tok_ref[0]
    scale = jnp.where(tok == 1, 0.0, SCALE).astype(jnp.float32)
    for t in range(8):
        blk = y[:, 128 * t : 128 * (t + 1)] * scale[t][None, :]
        o_ref[0, :, t, :, :] = blk.reshape(4, 8, 128)


def _build_finish():
    return pl.pallas_call(
        _finish_body,
        grid=(N_L, 4),
        in_specs=[
            pl.BlockSpec((1, 1024, 4 * EMB_DIM), lambda l, t: (l, t, 0)),
            pl.BlockSpec((1, 8, 128), lambda l, t: (l, t, 0)),
        ],
        out_specs=pl.BlockSpec(
            (1, 4, 8, 8, 128), lambda l, t: (l, 0, t, 0, 0)
        ),
        out_shape=jax.ShapeDtypeStruct((N_L, 4, 32, 8, 128), jnp.float32),
        compiler_params=pltpu.CompilerParams(
            dimension_semantics=("parallel", "parallel")
        ),
    )


def kernel(tokens, table):
    n = N_L * N_B
    idx = tokens.T.reshape(1, n).astype(jnp.int32)
    tok3 = tokens.T.reshape(N_L, 32, 128).astype(jnp.int32)
    table_pad = jnp.pad(table, ((0, 0), (0, 96))).reshape(4 * 1000000, EMB_DIM)
    g = _build_gather()(table_pad, idx)
    g3 = g.reshape(N_L, N_B, 4 * EMB_DIM)
    out5 = _build_finish()(g3, tok3)
    return out5.transpose(2, 4, 0, 1, 3).reshape(N_B, N_L, EMB_DIM)
